# baseline (speedup 1.0000x reference)
import jax
import jax.numpy as jnp
from jax.experimental import pallas as pl
from jax.experimental.pallas import tpu as pltpu

_BLOCK_ROWS = 2048
_COL_SPLIT = 2


def _fused_lowrank_kernel(x_ref, win_ref, wout_ref, out_ref, h_ref):
    @pl.when(pl.program_id(1) == 0)
    def _compute_h():
        h_ref[...] = jnp.dot(x_ref[...], win_ref[...],
                             preferred_element_type=jnp.float32)

    out_ref[...] = jnp.dot(h_ref[...], wout_ref[...],
                           preferred_element_type=jnp.float32)


@jax.jit
def kernel(x, shared_in, shared_out):
    n_tokens, in_features = x.shape
    rank, out_features = shared_out.shape
    col_block = out_features // _COL_SPLIT

    grid = (n_tokens // _BLOCK_ROWS, _COL_SPLIT)
    return pl.pallas_call(
        _fused_lowrank_kernel,
        grid=grid,
        in_specs=[
            pl.BlockSpec((_BLOCK_ROWS, in_features), lambda i, j: (i, 0)),
            pl.BlockSpec((in_features, rank), lambda i, j: (0, 0)),
            pl.BlockSpec((rank, col_block), lambda i, j: (0, j)),
        ],
        out_specs=pl.BlockSpec((_BLOCK_ROWS, col_block), lambda i, j: (i, j)),
        out_shape=jax.ShapeDtypeStruct((n_tokens, out_features), jnp.float32),
        scratch_shapes=[pltpu.VMEM((_BLOCK_ROWS, rank), jnp.float32)],
        compiler_params=pltpu.CompilerParams(
            dimension_semantics=("arbitrary", "arbitrary"),
        ),
    )(x, shared_in, shared_out)
